# baseline (speedup 1.0000x reference)
import functools

import jax
import jax.numpy as jnp
from jax import lax
from jax.experimental import pallas as pl
from jax.experimental.pallas import tpu as pltpu
from jax.experimental.pallas import tpu_sc as plsc

N = 10000; E = 320000; DIN = 128; DOUT = 128; HID = 16; H1 = 8; H3 = 1; EDIM = 16; G = 64

NP = 10240
CH = 128
EPAD = 323584
ACCW = 48
LW = 32
RPT = NP // 16

_mesh = functools.partial(
    plsc.VectorSubcoreMesh,
    core_axis_name="c", subcore_axis_name="s", num_cores=2, num_subcores=16)


def _sc_loop_attr(dst_p, ea_p):
    ept = EPAD // 32
    ch0 = 128
    nch0 = ept // ch0

    @functools.partial(
        pl.kernel,
        out_type=[jax.ShapeDtypeStruct((2, NP, EDIM), jnp.float32),
                  jax.ShapeDtypeStruct((2, NP, 16), jnp.float32)],
        mesh=_mesh(),
        compiler_params=pltpu.CompilerParams(needs_layout_passes=False),
        scratch_types=[
            pltpu.VMEM_SHARED((NP, EDIM), jnp.float32),
            pltpu.VMEM_SHARED((NP, 16), jnp.float32),
            pltpu.VMEM((2, ch0), jnp.int32),
            pltpu.VMEM((2, ch0, EDIM), jnp.float32),
            pltpu.VMEM((ch0, 16), jnp.float32),
            pltpu.SemaphoreType.DMA,
            pltpu.SemaphoreType.DMA,
        ],
    )
    def k(dst_hbm, ea_hbm, outa_hbm, outb_hbm, acca_sh, accb_sh, dst_v, pay,
          ones, sl0, sl1):
        c = lax.axis_index("c")
        s = lax.axis_index("s")
        slds = (sl0, sl1)

        zv = jnp.zeros((16,), jnp.float32)
        onehot = jnp.where(lax.iota(jnp.int32, 16) == 0, 1.0, 0.0).astype(jnp.float32)

        def zrow(j, _):
            ones[j, pl.ds(0, 16)] = zv
            return 0
        lax.fori_loop(0, ch0, zrow, 0)
        rpt = NP // 16
        for r in range((rpt + ch0 - 1) // ch0):
            base = jnp.minimum(s * rpt + r * ch0, NP - ch0)
            pltpu.sync_copy(ones, accb_sh.at[pl.ds(base, ch0)])
            pltpu.sync_copy(ones, acca_sh.at[pl.ds(base, ch0)])

        def orow(j, _):
            ones[j, pl.ds(0, 16)] = onehot
            return 0
        lax.fori_loop(0, ch0, orow, 0)
        plsc.subcore_barrier()

        w = c * 16 + s

        def issue_loads(i, b):
            off = w * ept + jnp.minimum(i, nch0 - 1) * ch0
            pltpu.async_copy(dst_hbm.at[pl.ds(off, ch0)], dst_v.at[b], slds[b])
            pltpu.async_copy(ea_hbm.at[pl.ds(off, ch0)], pay.at[b], slds[b])

        def step(b):
            pltpu.make_async_copy(dst_hbm.at[pl.ds(0, ch0)], dst_v.at[b],
                                  slds[b]).wait()
            pltpu.make_async_copy(ea_hbm.at[pl.ds(0, ch0)], pay.at[b],
                                  slds[b]).wait()

        issue_loads(0, 0)

        def pairc(o, _):
            step(0)
            issue_loads(2 * o + 1, 1)
            pltpu.sync_copy(pay.at[0], acca_sh.at[dst_v.at[0]], add=True)
            pltpu.sync_copy(ones, accb_sh.at[dst_v.at[0]], add=True)
            step(1)
            issue_loads(2 * o + 2, 0)
            pltpu.sync_copy(pay.at[1], acca_sh.at[dst_v.at[1]], add=True)
            pltpu.sync_copy(ones, accb_sh.at[dst_v.at[1]], add=True)
            return 0
        lax.fori_loop(0, nch0 // 2, pairc, 0)
        step(0)
        pltpu.sync_copy(pay.at[0], acca_sh.at[dst_v.at[0]], add=True)
        pltpu.sync_copy(ones, accb_sh.at[dst_v.at[0]], add=True)

        plsc.subcore_barrier()
        for r in range((rpt + ch0 - 1) // ch0):
            base = jnp.minimum(s * rpt + r * ch0, NP - ch0)
            sl = pl.ds(base, ch0)
            pltpu.sync_copy(acca_sh.at[sl], outa_hbm.at[c].at[sl])
            pltpu.sync_copy(accb_sh.at[sl], outb_hbm.at[c].at[sl])

    return k(dst_p, ea_p)


def _sc_edge(src_p, dst_p, ae, asrc_t, adst_t, xp_slab):
    ept = EPAD // 8
    art = 2 * NP // 16
    nch = ept // CH
    last = nch - 1

    @functools.partial(
        pl.kernel,
        out_type=jax.ShapeDtypeStruct((2, 2 * NP, ACCW), jnp.float32),
        mesh=_mesh(),
        compiler_params=pltpu.CompilerParams(
            needs_layout_passes=False, use_tc_tiling_on_sc=False),
        scratch_types=[
            pltpu.VMEM_SHARED((2 * NP, ACCW), jnp.float32),
            pltpu.VMEM((NP * 2,), jnp.float32),
            pltpu.VMEM((NP * 2,), jnp.float32),
            pltpu.VMEM((2, CH), jnp.int32),
            pltpu.VMEM((2, CH), jnp.int32),
            pltpu.VMEM((2, CH), jnp.int32),
            pltpu.VMEM((2, CH), jnp.int32),
            pltpu.VMEM((2, CH), jnp.int32),
            pltpu.VMEM((2, 2, CH), jnp.float32),
            pltpu.VMEM((2, CH * 2 + 16), jnp.float32),
            pltpu.VMEM((2, CH, 32), jnp.float32),
            pltpu.VMEM((2, CH, ACCW), jnp.float32),
            pltpu.SemaphoreType.DMA,
            pltpu.SemaphoreType.DMA,
            pltpu.SemaphoreType.DMA,
            pltpu.SemaphoreType.DMA,
            pltpu.SemaphoreType.DMA,
            pltpu.SemaphoreType.DMA,
        ],
    )
    def k(src_hbm, dst_hbm, ae_hbm, asrc_hbm, adst_hbm, xp_hbm, out_hbm,
          acc_sh, asrc_l, adst_l, src_v, dst_v, xoff_v, doff_v, sdoff_v, ae_v,
          ex_v, xbuf, pay, sl0, sl1, sg0, sg1, ss0, ss1):
        c = lax.axis_index("c")
        s = lax.axis_index("s")
        hp = lax.rem(s, 2)
        octant = lax.div(s, 2)
        q = c * 2 + hp
        h0 = q * 2
        slds = (sl0, sl1)
        sgxs = (sg0, sg1)
        pltpu.sync_copy(asrc_hbm.at[h0], asrc_l.at[pl.ds(0, NP)])
        pltpu.sync_copy(asrc_hbm.at[h0 + 1], asrc_l.at[pl.ds(NP, NP)])
        pltpu.sync_copy(adst_hbm.at[h0], adst_l.at[pl.ds(0, NP)])
        pltpu.sync_copy(adst_hbm.at[h0 + 1], adst_l.at[pl.ds(NP, NP)])

        zv = jnp.zeros((16,), jnp.float32)

        def zrow(j, _):
            for t in range(ACCW // 16):
                pay[0, j, pl.ds(t * 16, 16)] = zv
            return 0
        lax.fori_loop(0, CH, zrow, 0)
        for r in range(art // CH):
            pltpu.sync_copy(pay.at[0], acc_sh.at[pl.ds(s * art + r * CH, CH)])
        plsc.subcore_barrier()

        iota16 = lax.iota(jnp.int32, 16)

        def issue_loads(i, b):
            off = octant * ept + jnp.minimum(i, last) * CH
            pltpu.async_copy(src_hbm.at[pl.ds(off, CH)], src_v.at[b], slds[b])
            pltpu.async_copy(dst_hbm.at[pl.ds(off, CH)], dst_v.at[b], slds[b])
            pltpu.async_copy(ae_hbm.at[h0].at[pl.ds(off, CH)],
                             ae_v.at[b].at[0], slds[b])
            pltpu.async_copy(ae_hbm.at[h0 + 1].at[pl.ds(off, CH)],
                             ae_v.at[b].at[1], slds[b])

        def wait_loads(b):
            pltpu.make_async_copy(src_hbm.at[pl.ds(0, CH)], src_v.at[b],
                                  slds[b]).wait()
            pltpu.make_async_copy(dst_hbm.at[pl.ds(0, CH)], dst_v.at[b],
                                  slds[b]).wait()
            pltpu.make_async_copy(ae_hbm.at[0].at[pl.ds(0, CH)],
                                  ae_v.at[b].at[0], slds[b]).wait()
            pltpu.make_async_copy(ae_hbm.at[0].at[pl.ds(0, CH)],
                                  ae_v.at[b].at[1], slds[b]).wait()

        def front(i, b):
            wait_loads(b)
            for g in range(CH // 16):
                sl = pl.ds(g * 16, 16)
                sidx = src_v[b, sl]
                didx = dst_v[b, sl]
                rows2 = (iota16 + g * 16) * 2
                for hh in range(2):
                    a = (plsc.load_gather(asrc_l, [sidx + hh * NP])
                         + plsc.load_gather(adst_l, [didx + hh * NP])
                         + ae_v[b, hh, sl])
                    a = jnp.maximum(a, 0.2 * a)
                    plsc.store_scatter(ex_v.at[b], [rows2 + hh], jnp.exp(a))
                xoff_v[b, sl] = sidx + q * NP
                doff_v[b, sl] = didx + hp * NP
            pltpu.async_copy(xp_hbm.at[xoff_v.at[b]], xbuf.at[b], sgxs[b])
            issue_loads(i + 1, 1 - b)

        def back_issue(b):
            pltpu.make_async_copy(xp_hbm.at[pl.ds(0, CH)], xbuf.at[b],
                                  sgxs[b]).wait()

            def gbody(g, _):
                sl = pl.ds(g * 16, 16)
                sdoff_v[b, sl] = doff_v[b, sl]
                for jj in range(16):
                    row = g * 16 + jj
                    ev = ex_v[b, pl.ds(2 * row, 16)]
                    e0 = ev[0]
                    e1 = ev[1]
                    pay[b, row, pl.ds(0, 16)] = e0 * xbuf[b, row, pl.ds(0, 16)]
                    pay[b, row, pl.ds(16, 16)] = e1 * xbuf[b, row, pl.ds(16, 16)]
                    pay[b, row, pl.ds(32, 16)] = ev
                return 0
            lax.fori_loop(0, CH // 16, gbody, 0)
            return pltpu.async_copy(pay.at[b], acc_sh.at[sdoff_v.at[b]],
                                    (ss0, ss1)[b], add=True)

        issue_loads(0, 0)
        front(0, 0)

        def pair(o, _):
            front(2 * o + 1, 1)
            h0 = back_issue(0)
            front(2 * o + 2, 0)
            h1 = back_issue(1)
            h0.wait()
            h1.wait()
            return 0
        lax.fori_loop(0, nch // 2, pair, 0)
        pltpu.make_async_copy(xp_hbm.at[pl.ds(0, CH)], xbuf.at[0],
                              sgxs[0]).wait()
        wait_loads(1)

        plsc.subcore_barrier()
        for r in range(art // CH):
            sl = pl.ds(s * art + r * CH, CH)
            pltpu.sync_copy(acc_sh.at[sl], out_hbm.at[c].at[sl])

    return k(src_p, dst_p, ae, asrc_t, adst_t, xp_slab)


def _tc_loop_finish(acca, accb):
    def body(a_ref, b_ref, o_ref):
        ea = a_ref[0] + a_ref[1]
        cnt = b_ref[0, :, 0:1] + b_ref[1, :, 0:1]
        o_ref[...] = ea / jnp.maximum(cnt, 1.0)

    bm = NP // 10
    return pl.pallas_call(
        body,
        grid=(10,),
        in_specs=[pl.BlockSpec((2, bm, EDIM), lambda i: (0, i, 0)),
                  pl.BlockSpec((2, bm, 16), lambda i: (0, i, 0))],
        out_specs=pl.BlockSpec((bm, EDIM), lambda i: (i, 0)),
        out_shape=jax.ShapeDtypeStruct((NP, EDIM), jnp.float32),
    )(acca, accb)


def _tc_edge_prep(ea_p, we1, me1, we2, me2, we3, me3):
    def body(ea_ref, w1_ref, m1_ref, w2_ref, m2_ref, w3_ref, m3_ref,
             o1_ref, o2_ref, o3_ref):
        ea = ea_ref[...]
        for w_ref, m_ref, o_ref in ((w1_ref, m1_ref, o1_ref),
                                    (w2_ref, m2_ref, o2_ref),
                                    (w3_ref, m3_ref, o3_ref)):
            wm = jnp.dot(w_ref[...], m_ref[...], preferred_element_type=jnp.float32)
            o_ref[...] = lax.dot_general(
                wm, ea, (((0,), (1,)), ((), ())),
                preferred_element_type=jnp.float32)

    bm = 4096
    wspec = pl.BlockSpec((EDIM, 128), lambda i: (0, 0))
    mspec = pl.BlockSpec((128, 8), lambda i: (0, 0))
    ospec = pl.BlockSpec((8, bm), lambda i: (0, i))
    outs = pl.pallas_call(
        body,
        grid=(EPAD // bm,),
        in_specs=[pl.BlockSpec((bm, EDIM), lambda i: (i, 0)),
                  wspec, mspec, wspec, mspec, wspec, mspec],
        out_specs=[ospec, ospec, ospec],
        out_shape=[jax.ShapeDtypeStruct((8, EPAD), jnp.float32)] * 3,
    )(ea_p, we1, me1, we2, me2, we3, me3)
    return outs


def _tc_prep(h, w, msrc, mdst, la, we, me):
    def body(h_ref, w_ref, ms_ref, md_ref, la_ref, we_ref, me_ref,
             xp_ref, xps_ref, as_ref, ad_ref, exl_ref):
        xp = jnp.dot(h_ref[...], w_ref[...], preferred_element_type=jnp.float32)
        asrc = jnp.dot(xp, ms_ref[...], preferred_element_type=jnp.float32)
        adst = jnp.dot(xp, md_ref[...], preferred_element_type=jnp.float32)
        wm = jnp.dot(we_ref[...], me_ref[...], preferred_element_type=jnp.float32)
        aeloop = jnp.dot(la_ref[...], wm, preferred_element_type=jnp.float32)
        al = asrc + adst + aeloop
        al = jnp.maximum(al, 0.2 * al)
        exl_ref[...] = jnp.exp(al)
        xp_ref[...] = xp
        for qq in range(4):
            xps_ref[qq] = xp[:, qq * 32:(qq + 1) * 32]
        as_ref[...] = lax.dot_general(
            ms_ref[...], xp, (((0,), (1,)), ((), ())),
            preferred_element_type=jnp.float32)
        ad_ref[...] = lax.dot_general(
            md_ref[...], xp, (((0,), (1,)), ((), ())),
            preferred_element_type=jnp.float32)

    bm = NP // 10
    return pl.pallas_call(
        body,
        grid=(10,),
        in_specs=[pl.BlockSpec((bm, 128), lambda i: (i, 0)),
                  pl.BlockSpec((128, 128), lambda i: (0, 0)),
                  pl.BlockSpec((128, 8), lambda i: (0, 0)),
                  pl.BlockSpec((128, 8), lambda i: (0, 0)),
                  pl.BlockSpec((bm, EDIM), lambda i: (i, 0)),
                  pl.BlockSpec((EDIM, 128), lambda i: (0, 0)),
                  pl.BlockSpec((128, 8), lambda i: (0, 0))],
        out_specs=[pl.BlockSpec((bm, 128), lambda i: (i, 0)),
                   pl.BlockSpec((4, bm, 32), lambda i: (0, i, 0)),
                   pl.BlockSpec((8, bm), lambda i: (0, i)),
                   pl.BlockSpec((8, bm), lambda i: (0, i)),
                   pl.BlockSpec((bm, 8), lambda i: (i, 0))],
        out_shape=[jax.ShapeDtypeStruct((NP, 128), jnp.float32),
                   jax.ShapeDtypeStruct((4, NP, 32), jnp.float32),
                   jax.ShapeDtypeStruct((8, NP), jnp.float32),
                   jax.ShapeDtypeStruct((8, NP), jnp.float32),
                   jax.ShapeDtypeStruct((NP, 8), jnp.float32)],
    )(h, w, msrc, mdst, la, we, me)


def _tc_norm(acc, exl, xp_sc, b, k4, do_elu):
    def body(acc_ref, exl_ref, xp_ref, b_ref, k4_ref, o_ref):
        k4 = k4_ref[...]
        for sc in range(2):
            exl_sc = exl_ref[:, sc * 4:(sc + 1) * 4]
            e64 = jnp.dot(exl_sc, k4, preferred_element_type=jnp.float32)
            xp = xp_ref[:, sc * 64:(sc + 1) * 64]
            num = jnp.concatenate([acc_ref[sc, 0, :, :32],
                                   acc_ref[sc, 1, :, :32]], axis=1)
            num = num + e64 * xp
            den4 = jnp.concatenate([acc_ref[sc, 0, :, 32:34],
                                    acc_ref[sc, 1, :, 32:34]], axis=1) + exl_sc
            den = jnp.dot(den4, k4, preferred_element_type=jnp.float32)
            o = num / den + b_ref[0, sc * 64:(sc + 1) * 64]
            if do_elu:
                o = jnp.where(o > 0, o, jnp.exp(jnp.minimum(o, 0.0)) - 1.0)
            o_ref[:, sc * 64:(sc + 1) * 64] = o

    bm = NP // 10
    return pl.pallas_call(
        body,
        grid=(10,),
        in_specs=[pl.BlockSpec((2, 2, bm, ACCW), lambda i: (0, 0, i, 0)),
                  pl.BlockSpec((bm, 8), lambda i: (i, 0)),
                  pl.BlockSpec((bm, 128), lambda i: (i, 0)),
                  pl.BlockSpec((1, 128), lambda i: (0, 0)),
                  pl.BlockSpec((4, 64), lambda i: (0, 0))],
        out_specs=pl.BlockSpec((bm, 128), lambda i: (i, 0)),
        out_shape=jax.ShapeDtypeStruct((NP, 128), jnp.float32),
    )(acc, exl, xp_sc, b, k4)


def _tc_pool(h3, p):
    def body(p_ref, h_ref, o_ref):
        pm = p_ref[...]
        s = lax.dot_general(pm, h_ref[...], (((0,), (0,)), ((), ())),
                            preferred_element_type=jnp.float32)
        cnt = jnp.sum(pm, axis=0)[:, None]
        o_ref[...] = s / jnp.maximum(cnt, 1.0)

    return pl.pallas_call(
        body,
        in_specs=[pl.BlockSpec((NP, G), lambda: (0, 0)),
                  pl.BlockSpec((NP, 128), lambda: (0, 0))],
        out_specs=pl.BlockSpec((G, 128), lambda: (0, 0)),
        out_shape=jax.ShapeDtypeStruct((G, 128), jnp.float32),
    )(p, h3)


def _mask8(a):
    return (jnp.eye(8, dtype=jnp.float32)[:, None, :] * a[0][:, :, None]).reshape(128, 8)


def _mask1(a):
    return jnp.tile(a[0, 0][:, None], (1, 8))


def kernel(x, edge_index, edge_attr, batch, W1, a_src1, a_dst1, We1, a_e1, b1,
           W2, a_src2, a_dst2, We2, a_e2, b2, W3, a_src3, a_dst3, We3, a_e3, b3):
    f32 = jnp.float32
    src = edge_index[0]
    dst = edge_index[1]
    epad = EPAD - E
    src_p = jnp.concatenate([src, jnp.full((epad,), N, jnp.int32)])
    dst_p = jnp.concatenate([dst, jnp.full((epad,), N, jnp.int32)])
    ea_p = jnp.concatenate([edge_attr, jnp.zeros((epad, EDIM), f32)])
    x_p = jnp.concatenate([x, jnp.zeros((NP - N, DIN), f32)])

    p = (batch[:, None] == jnp.arange(G, dtype=jnp.int32)[None, :]).astype(f32)
    p = jnp.concatenate([p, jnp.zeros((NP - N, G), f32)])

    k4 = (jnp.eye(4, dtype=f32)[:, :, None] * jnp.ones((1, 1, 16), f32)).reshape(4, 64)

    msrc = (_mask8(a_src1), _mask8(a_src2), _mask1(a_src3))
    mdst = (_mask8(a_dst1), _mask8(a_dst2), _mask1(a_dst3))
    me = (_mask8(a_e1), _mask8(a_e2), _mask1(a_e3))
    ws = (W1, W2, W3)
    wes = (We1, We2, We3)
    bs = (b1.reshape(1, 128), b2.reshape(1, 128), b3.reshape(1, 128))

    acca, accb = _sc_loop_attr(dst_p, ea_p)
    la = _tc_loop_finish(acca, accb)
    aes = _tc_edge_prep(ea_p, We1, me[0], We2, me[1], We3, me[2])

    h = x_p
    for l in range(3):
        xp, xps, asrc_t, adst_t, exl = _tc_prep(h, ws[l], msrc[l], mdst[l], la,
                                                wes[l], me[l])
        acc = _sc_edge(src_p, dst_p, aes[l], asrc_t, adst_t,
                       xps.reshape(4 * NP, 32))
        h = _tc_norm(acc.reshape(2, 2, NP, ACCW), exl, xp, bs[l], k4,
                     do_elu=(l < 2))

    out = _tc_pool(h, p)
    return out
